# baseline (speedup 1.0000x reference)
import jax
import jax.numpy as jnp
from jax.experimental import pallas as pl
from jax.experimental.pallas import tpu as pltpu

_N, _F, _T = 32, 360, 2048
_TT = 512


def _ce_body(out_ref, oh_ref, acc_ref):
    i = pl.program_id(0)
    j = pl.program_id(1)

    x = out_ref[0]
    oh = oh_ref[0]
    l = jnp.log(x + 1e-20)

    s_prod = jnp.sum(oh * l)
    colsum = jnp.sum(l, axis=0)
    zero_col = jnp.all(oh == 0, axis=0)
    corr = jnp.sum(jnp.where(zero_col, colsum, 0.0))
    step = s_prod + corr * (1.0 / _F)

    @pl.when(jnp.logical_and(i == 0, j == 0))
    def _():
        acc_ref[0, 0] = 0.0

    acc_ref[0, 0] += step


def kernel(output, one_hot):
    out = jnp.reshape(output, (_N, _F, _T))
    acc = pl.pallas_call(
        _ce_body,
        grid=(_N, _T // _TT),
        in_specs=[
            pl.BlockSpec((1, _F, _TT), lambda i, j: (i, 0, j)),
            pl.BlockSpec((1, _F, _TT), lambda i, j: (i, 0, j)),
        ],
        out_specs=pl.BlockSpec((1, 1), lambda i, j: (0, 0),
                               memory_space=pltpu.SMEM),
        out_shape=jax.ShapeDtypeStruct((1, 1), jnp.float32),
    )(out, one_hot)
    return -acc[0, 0] / (_T * _N)


# speedup vs baseline: 1.7690x; 1.7690x over previous
import jax
import jax.numpy as jnp
from jax.experimental import pallas as pl
from jax.experimental.pallas import tpu as pltpu

_N, _F, _T = 32, 360, 2048
_TT = 2048


def _ce_body(out_ref, oh_ref, acc_ref):
    i = pl.program_id(0)
    j = pl.program_id(1)

    x = out_ref[0]
    oh = oh_ref[0]
    l = jnp.log(x + 1e-20)

    s_prod = jnp.sum(oh * l)
    colsum = jnp.sum(l, axis=0)
    zero_col = jnp.all(oh == 0, axis=0)
    corr = jnp.sum(jnp.where(zero_col, colsum, 0.0))
    step = s_prod + corr * (1.0 / _F)

    @pl.when(jnp.logical_and(i == 0, j == 0))
    def _():
        acc_ref[0, 0] = 0.0

    acc_ref[0, 0] += step


def kernel(output, one_hot):
    out = jnp.reshape(output, (_N, _F, _T))
    acc = pl.pallas_call(
        _ce_body,
        grid=(_N, _T // _TT),
        in_specs=[
            pl.BlockSpec((1, _F, _TT), lambda i, j: (i, 0, j)),
            pl.BlockSpec((1, _F, _TT), lambda i, j: (i, 0, j)),
        ],
        out_specs=pl.BlockSpec((1, 1), lambda i, j: (0, 0),
                               memory_space=pltpu.SMEM),
        out_shape=jax.ShapeDtypeStruct((1, 1), jnp.float32),
    )(out, one_hot)
    return -acc[0, 0] / (_T * _N)
